# baseline (speedup 1.0000x reference)
import functools

import jax
import jax.numpy as jnp
from jax.experimental import pallas as pl
from jax.experimental.pallas import tpu as pltpu

MAX_LEN = 512
INFLIGHT = 32
N_PAD = 1024


def _dma_kernel(t_ref, out_ref, t8_ref, sem, *, seq_len_q, seq_len_k, max_len,
                inflight):
    n_groups = 8
    rows_per_group = seq_len_q // n_groups
    tv = t_ref[...]

    def wait_one():
        pltpu.make_async_copy(
            t_ref.at[pl.ds(0, seq_len_k), :], out_ref.at[0], sem).wait()

    for c in range(n_groups):
        if c > 0:
            t8_ref[c - 1] = pltpu.roll(tv, N_PAD - c, 0)

        def body(m, carry, c=c):
            i = n_groups * m + (n_groups - 1 - c)
            s = (max_len - 1) - i
            aligned = pl.multiple_of(s - c, 8)
            src = (t_ref.at[pl.ds(aligned, seq_len_k), :] if c == 0
                   else t8_ref.at[c - 1, pl.ds(aligned, seq_len_k), :])
            pltpu.make_async_copy(src, out_ref.at[i], sem).start()
            if c == 0:
                @pl.when(m >= inflight)
                def _():
                    wait_one()
            else:
                wait_one()
            return carry

        jax.lax.fori_loop(0, rows_per_group, body, 0)

    for _ in range(inflight):
        wait_one()


def kernel(q, k, rel_pos_enc):
    seq_len_q = q.shape[1]
    seq_len_k = k.shape[1]
    d = rel_pos_enc.shape[1]
    n = rel_pos_enc.shape[0]
    padded = jnp.pad(rel_pos_enc, ((0, N_PAD - n), (0, 0)))

    body = functools.partial(
        _dma_kernel,
        seq_len_q=seq_len_q,
        seq_len_k=seq_len_k,
        max_len=MAX_LEN,
        inflight=INFLIGHT,
    )
    return pl.pallas_call(
        body,
        in_specs=[
            pl.BlockSpec(memory_space=pltpu.MemorySpace.VMEM),
        ],
        out_specs=pl.BlockSpec(memory_space=pltpu.MemorySpace.HBM),
        out_shape=jax.ShapeDtypeStruct((seq_len_q, seq_len_k, d), rel_pos_enc.dtype),
        scratch_shapes=[
            pltpu.VMEM((7, N_PAD, d), rel_pos_enc.dtype),
            pltpu.SemaphoreType.DMA,
        ],
    )(padded)


# speedup vs baseline: 1.0111x; 1.0111x over previous
import functools

import jax
import jax.numpy as jnp
from jax.experimental import pallas as pl
from jax.experimental.pallas import tpu as pltpu

MAX_LEN = 512
INFLIGHT = 16
N_PAD = 1024


def _dma_kernel(t_ref, out_ref, t8_ref, sem, *, seq_len_q, seq_len_k, max_len,
                inflight):
    n_groups = 8
    rows_per_group = seq_len_q // n_groups
    tv = t_ref[...]

    def wait_one():
        pltpu.make_async_copy(
            t_ref.at[pl.ds(0, seq_len_k), :], out_ref.at[0], sem).wait()

    for c in range(n_groups):
        if c > 0:
            t8_ref[c - 1] = pltpu.roll(tv, N_PAD - c, 0)

        def body(m, carry, c=c):
            i = n_groups * m + (n_groups - 1 - c)
            s = (max_len - 1) - i
            aligned = pl.multiple_of(s - c, 8)
            src = (t_ref.at[pl.ds(aligned, seq_len_k), :] if c == 0
                   else t8_ref.at[c - 1, pl.ds(aligned, seq_len_k), :])
            pltpu.make_async_copy(src, out_ref.at[i], sem).start()
            if c == 0:
                @pl.when(m >= inflight)
                def _():
                    wait_one()
            else:
                wait_one()
            return carry

        jax.lax.fori_loop(0, rows_per_group, body, 0)

    for _ in range(inflight):
        wait_one()


def kernel(q, k, rel_pos_enc):
    seq_len_q = q.shape[1]
    seq_len_k = k.shape[1]
    d = rel_pos_enc.shape[1]
    n = rel_pos_enc.shape[0]
    padded = jnp.pad(rel_pos_enc, ((0, N_PAD - n), (0, 0)))

    body = functools.partial(
        _dma_kernel,
        seq_len_q=seq_len_q,
        seq_len_k=seq_len_k,
        max_len=MAX_LEN,
        inflight=INFLIGHT,
    )
    return pl.pallas_call(
        body,
        in_specs=[
            pl.BlockSpec(memory_space=pltpu.MemorySpace.VMEM),
        ],
        out_specs=pl.BlockSpec(memory_space=pltpu.MemorySpace.HBM),
        out_shape=jax.ShapeDtypeStruct((seq_len_q, seq_len_k, d), rel_pos_enc.dtype),
        scratch_shapes=[
            pltpu.VMEM((7, N_PAD, d), rel_pos_enc.dtype),
            pltpu.SemaphoreType.DMA,
        ],
    )(padded)
